# baseline (speedup 1.0000x reference)
import functools

import jax
import jax.numpy as jnp
from jax import lax
from jax.experimental import pallas as pl
from jax.experimental.pallas import tpu as pltpu
from jax.experimental.pallas import tpu_sc as plsc

B = 4
S = 2048
D = 1024
SCALE = 32.0

_INFO = plsc.get_sparse_core_info()
NC = _INFO.num_cores
NS = _INFO.num_subcores
NW = NC * NS
TOK_PER_W = (B * S) // NW
CHUNK = 16
NCHUNK = TOK_PER_W // CHUNK
LANES = 16
JSTEPS = D // LANES


def _body(ids_hbm, tok_hbm, pos_hbm, out_hbm, idx_v, rows_v, pos_v, gsem, psem):
    wid = lax.axis_index("s") * NC + lax.axis_index("c")
    per_row = S // TOK_PER_W
    b = wid // per_row
    col0 = (wid % per_row) * TOK_PER_W

    pltpu.sync_copy(ids_hbm.at[b, pl.ds(col0, TOK_PER_W)], idx_v)

    for c in range(NCHUNK):
        s0 = col0 + c * CHUNK
        gcp = pltpu.async_copy(
            tok_hbm.at[idx_v.at[pl.ds(c * CHUNK, CHUNK)]], rows_v, gsem)
        pcp = pltpu.async_copy(pos_hbm.at[pl.ds(s0, CHUNK), :], pos_v, psem)
        gcp.wait()
        pcp.wait()

        def jloop(j, r):
            sl = pl.ds(j * LANES, LANES)
            rows_v[r, sl] = rows_v[r, sl] * SCALE + pos_v[r, sl]
            return r

        def rloop(r, carry):
            lax.fori_loop(0, JSTEPS, jloop, r, unroll=4)
            return carry

        lax.fori_loop(0, CHUNK, rloop, 0)

        pltpu.sync_copy(rows_v, out_hbm.at[b, pl.ds(s0, CHUNK), :])


@functools.partial(jax.jit, static_argnames=())
def kernel(input_ids, token_table, pos_table):
    mesh = plsc.VectorSubcoreMesh(core_axis_name="c", subcore_axis_name="s")
    run = pl.kernel(
        _body,
        mesh=mesh,
        out_type=jax.ShapeDtypeStruct((B, S, D), jnp.float32),
        scratch_types=[
            pltpu.VMEM((TOK_PER_W,), jnp.int32),
            pltpu.VMEM((CHUNK, D), jnp.float32),
            pltpu.VMEM((CHUNK, D), jnp.float32),
            pltpu.SemaphoreType.DMA,
            pltpu.SemaphoreType.DMA,
        ],
    )
    return run(input_ids.astype(jnp.int32), token_table, pos_table)
